# baseline (speedup 1.0000x reference)
import jax
import jax.numpy as jnp
from jax.experimental import pallas as pl

T = 8192
H = 2048
E = 64
TB = 1024
TBT = 2048


def _router_body(x_ref, w_ref, b_ref, out_ref):
    x = x_ref[...]
    w = w_ref[...]
    lt = jax.lax.dot_general(
        w, x,
        dimension_numbers=(((1,), (1,)), ((), ())),
        preferred_element_type=jnp.float32,
    ) + b_ref[...]
    row = jax.lax.broadcasted_iota(jnp.int32, lt.shape, 0)
    m1 = jnp.max(lt, axis=0)
    i1 = jnp.min(jnp.where(lt == m1, row, E), axis=0)
    first1 = row == i1[None, :]
    masked = jnp.where(first1, -jnp.inf, lt)
    m2 = jnp.max(masked, axis=0)
    i2 = jnp.min(jnp.where(masked == m2, row, E), axis=0)
    first2 = row == i2[None, :]
    r = jnp.exp(m2 - m1)
    denom = 1.0 + r
    p1 = 1.0 / denom
    p2 = r / denom
    out_ref[...] = jnp.where(first1, p1[None, :],
                             jnp.where(first2, p2[None, :], 0.0))


def _transpose_body(in_ref, out_ref):
    out_ref[...] = in_ref[...].T


def kernel(hidden_states, weight, bias):
    bias2d = bias.reshape(E, 1)
    scores_t = pl.pallas_call(
        _router_body,
        grid=(T // TB,),
        in_specs=[
            pl.BlockSpec((TB, H), lambda i: (i, 0)),
            pl.BlockSpec((E, H), lambda i: (0, 0)),
            pl.BlockSpec((E, 1), lambda i: (0, 0)),
        ],
        out_specs=pl.BlockSpec((E, TB), lambda i: (0, i)),
        out_shape=jax.ShapeDtypeStruct((E, T), jnp.float32),
    )(hidden_states, weight, bias2d)
    return pl.pallas_call(
        _transpose_body,
        grid=(T // TBT,),
        in_specs=[pl.BlockSpec((E, TBT), lambda i: (0, i))],
        out_specs=pl.BlockSpec((TBT, E), lambda i: (i, 0)),
        out_shape=jax.ShapeDtypeStruct((T, E), jnp.float32),
    )(scores_t)


# speedup vs baseline: 1.0773x; 1.0773x over previous
import jax
import jax.numpy as jnp
from jax.experimental import pallas as pl

T = 8192
H = 2048
E = 64
TB = 1024


def _router_body(x_ref, w_ref, b_ref, out_ref):
    x = x_ref[...]
    w = w_ref[...]
    lt = jax.lax.dot_general(
        w, x,
        dimension_numbers=(((1,), (1,)), ((), ())),
        preferred_element_type=jnp.float32,
    ) + b_ref[...]
    row = jax.lax.broadcasted_iota(jnp.int32, lt.shape, 0)
    m1 = jnp.max(lt, axis=0)
    i1 = jnp.min(jnp.where(lt == m1, row, E), axis=0)
    first1 = row == i1[None, :]
    masked = jnp.where(first1, -jnp.inf, lt)
    m2 = jnp.max(masked, axis=0)
    i2 = jnp.min(jnp.where(masked == m2, row, E), axis=0)
    r = jnp.exp(m2 - m1)
    denom = 1.0 + r
    p1 = 1.0 / denom
    p2 = r / denom
    idxT = jnp.stack([i1, i2], axis=0).T
    probT = jnp.stack([p1, p2], axis=0).T
    i1c = idxT[:, 0:1]
    i2c = idxT[:, 1:2]
    p1c = probT[:, 0:1]
    p2c = probT[:, 1:2]
    lane = jax.lax.broadcasted_iota(jnp.int32, (TB, E), 1)
    out_ref[...] = jnp.where(lane == i1c, p1c,
                             jnp.where(lane == i2c, p2c, 0.0))


def kernel(hidden_states, weight, bias):
    bias2d = bias.reshape(E, 1)
    return pl.pallas_call(
        _router_body,
        grid=(T // TB,),
        in_specs=[
            pl.BlockSpec((TB, H), lambda i: (i, 0)),
            pl.BlockSpec((E, H), lambda i: (0, 0)),
            pl.BlockSpec((E, 1), lambda i: (0, 0)),
        ],
        out_specs=pl.BlockSpec((TB, E), lambda i: (i, 0)),
        out_shape=jax.ShapeDtypeStruct((T, E), jnp.float32),
    )(hidden_states, weight, bias2d)


# speedup vs baseline: 1.1069x; 1.0274x over previous
import jax
import jax.numpy as jnp
from jax.experimental import pallas as pl

T = 8192
H = 2048
E = 64
TB = 2048


def _router_body(x_ref, w_ref, b_ref, out_ref):
    x = x_ref[...]
    w = w_ref[...]
    lt = jax.lax.dot_general(
        w, x,
        dimension_numbers=(((1,), (1,)), ((), ())),
        preferred_element_type=jnp.float32,
    ) + b_ref[...]
    row = jax.lax.broadcasted_iota(jnp.int32, lt.shape, 0)
    m1 = jnp.max(lt, axis=0)
    i1 = jnp.min(jnp.where(lt == m1, row, E), axis=0)
    first1 = row == i1[None, :]
    masked = jnp.where(first1, -jnp.inf, lt)
    m2 = jnp.max(masked, axis=0)
    i2 = jnp.min(jnp.where(masked == m2, row, E), axis=0)
    first2 = row == i2[None, :]
    r = jnp.exp(m2 - m1)
    denom = 1.0 + r
    p1 = 1.0 / denom
    p2 = r / denom
    out_t = jnp.where(first1, p1[None, :],
                      jnp.where(first2, p2[None, :], 0.0))
    out_ref[...] = out_t.T


def kernel(hidden_states, weight, bias):
    bias2d = bias.reshape(E, 1)
    return pl.pallas_call(
        _router_body,
        grid=(T // TB,),
        in_specs=[
            pl.BlockSpec((TB, H), lambda i: (i, 0)),
            pl.BlockSpec((E, H), lambda i: (0, 0)),
            pl.BlockSpec((E, 1), lambda i: (0, 0)),
        ],
        out_specs=pl.BlockSpec((TB, E), lambda i: (i, 0)),
        out_shape=jax.ShapeDtypeStruct((T, E), jnp.float32),
    )(hidden_states, weight, bias2d)


# speedup vs baseline: 1.1411x; 1.0309x over previous
import jax
import jax.numpy as jnp
from jax.experimental import pallas as pl

T = 8192
H = 2048
E = 64
TB = 1024


def _router_body(x_ref, w_ref, b_ref, out_ref):
    x = x_ref[...]
    w = w_ref[...]
    lt = jax.lax.dot_general(
        w, x,
        dimension_numbers=(((1,), (1,)), ((), ())),
        preferred_element_type=jnp.float32,
    ) + b_ref[...]
    row = jax.lax.broadcasted_iota(jnp.int32, lt.shape, 0)
    m1 = jnp.max(lt, axis=0)
    i1 = jnp.min(jnp.where(lt == m1, row, E), axis=0)
    first1 = row == i1[None, :]
    masked = jnp.where(first1, -jnp.inf, lt)
    m2 = jnp.max(masked, axis=0)
    i2 = jnp.min(jnp.where(masked == m2, row, E), axis=0)
    first2 = row == i2[None, :]
    r = jnp.exp(m2 - m1)
    denom = 1.0 + r
    p1 = 1.0 / denom
    p2 = r / denom
    out_t = jnp.where(first1, p1[None, :],
                      jnp.where(first2, p2[None, :], 0.0))
    out_ref[...] = out_t.T


def kernel(hidden_states, weight, bias):
    bias2d = bias.reshape(E, 1)
    return pl.pallas_call(
        _router_body,
        grid=(T // TB,),
        in_specs=[
            pl.BlockSpec((TB, H), lambda i: (i, 0)),
            pl.BlockSpec((E, H), lambda i: (0, 0)),
            pl.BlockSpec((E, 1), lambda i: (0, 0)),
        ],
        out_specs=pl.BlockSpec((TB, E), lambda i: (i, 0)),
        out_shape=jax.ShapeDtypeStruct((T, E), jnp.float32),
    )(hidden_states, weight, bias2d)
